# baseline (device time: 24798 ns/iter reference)
import jax
import jax.numpy as jnp
from jax import lax
from jax.experimental import pallas as pl
from jax.experimental.pallas import tpu as pltpu

N_DEV = 16


def kernel(x, w_mat):
    k_dim, m_per = x.shape
    _, n_dim = w_mat.shape
    blk = m_per

    def body(x_ref, w_ref, out_ref, xb, gathered, send_sems, recv_sems):
        my_pos = lax.axis_index("i")

        xb[...] = x_ref[...].astype(jnp.bfloat16)

        sends = []
        for d in range(1, N_DEV):
            peer = lax.rem(my_pos + d, N_DEV)
            rdma = pltpu.make_async_remote_copy(
                src_ref=xb.at[pl.ds(peer * blk, blk), :],
                dst_ref=gathered.at[:, pl.ds(my_pos * blk, blk)],
                send_sem=send_sems.at[d],
                recv_sem=recv_sems.at[my_pos],
                device_id=(peer,),
                device_id_type=pl.DeviceIdType.MESH,
            )
            rdma.start()
            sends.append(rdma)

        gathered[:, pl.ds(my_pos * blk, blk)] = xb[pl.ds(my_pos * blk, blk), :]

        for d in range(1, N_DEV):
            src = lax.rem(my_pos + d, N_DEV)
            recv = pltpu.make_async_remote_copy(
                src_ref=xb.at[pl.ds(src * blk, blk), :],
                dst_ref=gathered.at[:, pl.ds(src * blk, blk)],
                send_sem=send_sems.at[d],
                recv_sem=recv_sems.at[src],
                device_id=(src,),
                device_id_type=pl.DeviceIdType.MESH,
            )
            recv.wait_recv()

        acc = jnp.dot(
            gathered[...],
            w_ref[...].astype(jnp.bfloat16),
            preferred_element_type=jnp.float32,
        )
        out_ref[...] = acc * (1.0 / (1.0 + jnp.exp(-acc)))

        for rdma in sends:
            rdma.wait_send()

    return pl.pallas_call(
        body,
        out_shape=jax.ShapeDtypeStruct((m_per, n_dim), jnp.float32),
        in_specs=[
            pl.BlockSpec(memory_space=pltpu.VMEM),
            pl.BlockSpec(memory_space=pltpu.VMEM),
        ],
        out_specs=pl.BlockSpec(memory_space=pltpu.VMEM),
        scratch_shapes=[
            pltpu.VMEM((k_dim, m_per), jnp.bfloat16),
            pltpu.VMEM((m_per, k_dim), jnp.bfloat16),
            pltpu.SemaphoreType.DMA((N_DEV,)),
            pltpu.SemaphoreType.DMA((N_DEV,)),
        ],
    )(x, w_mat)


# device time: 21679 ns/iter; 1.1439x vs baseline; 1.1439x over previous
import jax
import jax.numpy as jnp
from jax import lax
from jax.experimental import pallas as pl
from jax.experimental.pallas import tpu as pltpu

N_DEV = 16
N_CHUNKS = 4
BLKS_PER_CHUNK = N_DEV // N_CHUNKS


def kernel(x, w_mat):
    k_dim, m_per = x.shape
    _, n_dim = w_mat.shape
    blk = m_per
    kc = k_dim // N_CHUNKS

    def body(x_ref, w_hbm, out_ref, xb, gathered, wbuf,
             send_sems, recv_sems, w_sems):
        my_pos = lax.axis_index("i")

        w_copies = []
        for c in range(N_CHUNKS):
            cp = pltpu.make_async_copy(
                w_hbm.at[pl.ds(c * kc, kc), :],
                wbuf.at[c],
                w_sems.at[c],
            )
            cp.start()
            w_copies.append(cp)

        xb[...] = x_ref[...].astype(jnp.bfloat16)

        barrier_sem = pltpu.get_barrier_semaphore()
        for d in range(1, N_DEV):
            peer = lax.rem(my_pos + d, N_DEV)
            pl.semaphore_signal(
                barrier_sem, inc=1,
                device_id=(peer,), device_id_type=pl.DeviceIdType.MESH,
            )
        pl.semaphore_wait(barrier_sem, N_DEV - 1)

        sends = []
        for d in range(1, N_DEV):
            peer = lax.rem(my_pos + d, N_DEV)
            rdma = pltpu.make_async_remote_copy(
                src_ref=xb.at[pl.ds(peer * blk, blk), :],
                dst_ref=gathered.at[:, pl.ds(my_pos * blk, blk)],
                send_sem=send_sems.at[d],
                recv_sem=recv_sems.at[my_pos],
                device_id=(peer,),
                device_id_type=pl.DeviceIdType.MESH,
            )
            rdma.start()
            sends.append(rdma)

        gathered[:, pl.ds(my_pos * blk, blk)] = xb[pl.ds(my_pos * blk, blk), :]

        acc = None
        for c in range(N_CHUNKS):
            for j in range(c * BLKS_PER_CHUNK, (c + 1) * BLKS_PER_CHUNK):
                recv = pltpu.make_async_remote_copy(
                    src_ref=xb.at[pl.ds(j * blk, blk), :],
                    dst_ref=gathered.at[:, pl.ds(j * blk, blk)],
                    send_sem=send_sems.at[0],
                    recv_sem=recv_sems.at[j],
                    device_id=(j,),
                    device_id_type=pl.DeviceIdType.MESH,
                )
                @pl.when(j != my_pos)
                def _(recv=recv):
                    recv.wait_recv()

            w_copies[c].wait()
            wc = wbuf[c].astype(jnp.bfloat16)
            partial = jnp.dot(
                gathered[:, pl.ds(c * kc, kc)],
                wc,
                preferred_element_type=jnp.float32,
            )
            acc = partial if acc is None else acc + partial

        out_ref[...] = acc * (1.0 / (1.0 + jnp.exp(-acc)))

        for rdma in sends:
            rdma.wait_send()

    return pl.pallas_call(
        body,
        out_shape=jax.ShapeDtypeStruct((m_per, n_dim), jnp.float32),
        in_specs=[
            pl.BlockSpec(memory_space=pltpu.VMEM),
            pl.BlockSpec(memory_space=pl.ANY),
        ],
        out_specs=pl.BlockSpec(memory_space=pltpu.VMEM),
        scratch_shapes=[
            pltpu.VMEM((k_dim, m_per), jnp.bfloat16),
            pltpu.VMEM((m_per, k_dim), jnp.bfloat16),
            pltpu.VMEM((N_CHUNKS, kc, n_dim), jnp.float32),
            pltpu.SemaphoreType.DMA((N_DEV,)),
            pltpu.SemaphoreType.DMA((N_DEV,)),
            pltpu.SemaphoreType.DMA((N_CHUNKS,)),
        ],
        compiler_params=pltpu.CompilerParams(collective_id=0),
    )(x, w_mat)


# device time: 18756 ns/iter; 1.3221x vs baseline; 1.1558x over previous
import jax
import jax.numpy as jnp
from jax import lax
from jax.experimental import pallas as pl
from jax.experimental.pallas import tpu as pltpu

N_DEV = 16
N_CHUNKS = 4
BLKS_PER_CHUNK = N_DEV // N_CHUNKS


def kernel(x, w_mat):
    k_dim, m_per = x.shape
    _, n_dim = w_mat.shape
    blk = m_per
    kc = k_dim // N_CHUNKS

    def body(x_hbm, w_hbm, out_hbm, x32, xb, gathered, xrow, wbuf, yout,
             send_sems, recv_sems, w_sems, x_sem, out_sem):
        my_pos = lax.axis_index("i")

        x_cp = pltpu.make_async_copy(x_hbm, x32, x_sem)
        x_cp.start()
        w_copies = []
        for c in range(N_CHUNKS):
            cp = pltpu.make_async_copy(
                w_hbm.at[pl.ds(c * kc, kc), :], wbuf.at[c], w_sems.at[c])
            cp.start()
            w_copies.append(cp)

        x_cp.wait()
        xb[...] = x32[...].astype(jnp.bfloat16)

        barrier_sem = pltpu.get_barrier_semaphore()
        for d in range(1, N_DEV):
            peer = lax.rem(my_pos + d, N_DEV)
            pl.semaphore_signal(
                barrier_sem, inc=1,
                device_id=(peer,), device_id_type=pl.DeviceIdType.MESH)
        pl.semaphore_wait(barrier_sem, N_DEV - 1)

        sends = []
        for d in range(1, N_DEV):
            peer = lax.rem(my_pos + d, N_DEV)
            rdma = pltpu.make_async_remote_copy(
                src_ref=xb.at[pl.ds(peer * blk, blk), :],
                dst_ref=gathered.at[my_pos],
                send_sem=send_sems.at[d],
                recv_sem=recv_sems.at[my_pos],
                device_id=(peer,),
                device_id_type=pl.DeviceIdType.MESH)
            rdma.start()
            sends.append(rdma)

        gathered[my_pos] = xb[pl.ds(my_pos * blk, blk), :]
        xrow[:, pl.ds(my_pos * blk, blk)] = gathered[my_pos]

        acc = None
        for c in range(N_CHUNKS):
            for j in range(c * BLKS_PER_CHUNK, (c + 1) * BLKS_PER_CHUNK):
                recv = pltpu.make_async_remote_copy(
                    src_ref=xb.at[pl.ds(j * blk, blk), :],
                    dst_ref=gathered.at[j],
                    send_sem=send_sems.at[0],
                    recv_sem=recv_sems.at[j],
                    device_id=(j,),
                    device_id_type=pl.DeviceIdType.MESH)

                @pl.when(j != my_pos)
                def _(recv=recv, j=j):
                    recv.wait_recv()
                    xrow[:, pl.ds(j * blk, blk)] = gathered[j]

            w_copies[c].wait()
            wc = wbuf[c].astype(jnp.bfloat16)
            partial = jnp.dot(
                xrow[:, pl.ds(c * kc, kc)], wc,
                preferred_element_type=jnp.float32)
            acc = partial if acc is None else acc + partial

        yout[...] = acc * (1.0 / (1.0 + jnp.exp(-acc)))
        out_cp = pltpu.make_async_copy(yout, out_hbm, out_sem)
        out_cp.start()
        out_cp.wait()

        for rdma in sends:
            rdma.wait_send()

    return pl.pallas_call(
        body,
        out_shape=jax.ShapeDtypeStruct((m_per, n_dim), jnp.float32),
        in_specs=[
            pl.BlockSpec(memory_space=pltpu.MemorySpace.HBM),
            pl.BlockSpec(memory_space=pltpu.MemorySpace.HBM),
        ],
        out_specs=pl.BlockSpec(memory_space=pltpu.MemorySpace.HBM),
        scratch_shapes=[
            pltpu.VMEM((k_dim, m_per), jnp.float32),
            pltpu.VMEM((k_dim, m_per), jnp.bfloat16),
            pltpu.VMEM((N_DEV, blk, blk), jnp.bfloat16),
            pltpu.VMEM((m_per, k_dim), jnp.bfloat16),
            pltpu.VMEM((N_CHUNKS, kc, n_dim), jnp.float32),
            pltpu.VMEM((m_per, n_dim), jnp.float32),
            pltpu.SemaphoreType.DMA((N_DEV,)),
            pltpu.SemaphoreType.DMA((N_DEV,)),
            pltpu.SemaphoreType.DMA((N_CHUNKS,)),
            pltpu.SemaphoreType.DMA,
            pltpu.SemaphoreType.DMA,
        ],
        compiler_params=pltpu.CompilerParams(collective_id=0),
    )(
        pltpu.with_memory_space_constraint(x, pltpu.MemorySpace.HBM),
        pltpu.with_memory_space_constraint(w_mat, pltpu.MemorySpace.HBM),
    )


# device time: 17151 ns/iter; 1.4459x vs baseline; 1.0936x over previous
import jax
import jax.numpy as jnp
from jax import lax
from jax.experimental import pallas as pl
from jax.experimental.pallas import tpu as pltpu

N_DEV = 16
N_CHUNKS = 8
BLKS_PER_CHUNK = N_DEV // N_CHUNKS


def kernel(x, w_mat):
    k_dim, m_per = x.shape
    _, n_dim = w_mat.shape
    blk = m_per
    kc = k_dim // N_CHUNKS

    def body(x_hbm, w_hbm, out_hbm, x32, xb, gathered, xrow, wbuf, yout,
             send_sems, recv_sems, w_sems, x_sem, out_sem):
        my_pos = lax.axis_index("i")

        x_cp = pltpu.make_async_copy(x_hbm, x32, x_sem)
        x_cp.start()
        w_copies = []
        for c in range(N_CHUNKS):
            cp = pltpu.make_async_copy(
                w_hbm.at[pl.ds(c * kc, kc), :], wbuf.at[c], w_sems.at[c])
            cp.start()
            w_copies.append(cp)

        barrier_sem = pltpu.get_barrier_semaphore()
        for d in range(1, N_DEV):
            peer = lax.rem(my_pos + d, N_DEV)
            pl.semaphore_signal(
                barrier_sem, inc=1,
                device_id=(peer,), device_id_type=pl.DeviceIdType.MESH)

        x_cp.wait()
        xb[...] = x32[...].astype(jnp.bfloat16)

        gathered[my_pos] = xb[pl.ds(my_pos * blk, blk), :]
        xrow[:, pl.ds(my_pos * blk, blk)] = gathered[my_pos]

        pl.semaphore_wait(barrier_sem, N_DEV - 1)

        sends = []
        for d in range(1, N_DEV):
            peer = lax.rem(my_pos + d, N_DEV)
            rdma = pltpu.make_async_remote_copy(
                src_ref=xb.at[pl.ds(peer * blk, blk), :],
                dst_ref=gathered.at[my_pos],
                send_sem=send_sems.at[d],
                recv_sem=recv_sems.at[my_pos],
                device_id=(peer,),
                device_id_type=pl.DeviceIdType.MESH)
            rdma.start()
            sends.append(rdma)

        acc = None
        for c in range(N_CHUNKS):
            for j in range(c * BLKS_PER_CHUNK, (c + 1) * BLKS_PER_CHUNK):
                recv = pltpu.make_async_remote_copy(
                    src_ref=xb.at[pl.ds(j * blk, blk), :],
                    dst_ref=gathered.at[j],
                    send_sem=send_sems.at[0],
                    recv_sem=recv_sems.at[j],
                    device_id=(j,),
                    device_id_type=pl.DeviceIdType.MESH)

                @pl.when(j != my_pos)
                def _(recv=recv, j=j):
                    recv.wait_recv()
                    xrow[:, pl.ds(j * blk, blk)] = gathered[j]

            w_copies[c].wait()
            wc = wbuf[c].astype(jnp.bfloat16)
            partial = jnp.dot(
                xrow[:, pl.ds(c * kc, kc)], wc,
                preferred_element_type=jnp.float32)
            acc = partial if acc is None else acc + partial

        yout[...] = acc * (1.0 / (1.0 + jnp.exp(-acc)))
        out_cp = pltpu.make_async_copy(yout, out_hbm, out_sem)
        out_cp.start()
        out_cp.wait()

        for rdma in sends:
            rdma.wait_send()

    return pl.pallas_call(
        body,
        out_shape=jax.ShapeDtypeStruct((m_per, n_dim), jnp.float32),
        in_specs=[
            pl.BlockSpec(memory_space=pltpu.MemorySpace.HBM),
            pl.BlockSpec(memory_space=pltpu.MemorySpace.HBM),
        ],
        out_specs=pl.BlockSpec(memory_space=pltpu.MemorySpace.HBM),
        scratch_shapes=[
            pltpu.VMEM((k_dim, m_per), jnp.float32),
            pltpu.VMEM((k_dim, m_per), jnp.bfloat16),
            pltpu.VMEM((N_DEV, blk, blk), jnp.bfloat16),
            pltpu.VMEM((m_per, k_dim), jnp.bfloat16),
            pltpu.VMEM((N_CHUNKS, kc, n_dim), jnp.float32),
            pltpu.VMEM((m_per, n_dim), jnp.float32),
            pltpu.SemaphoreType.DMA((N_DEV,)),
            pltpu.SemaphoreType.DMA((N_DEV,)),
            pltpu.SemaphoreType.DMA((N_CHUNKS,)),
            pltpu.SemaphoreType.DMA,
            pltpu.SemaphoreType.DMA,
        ],
        compiler_params=pltpu.CompilerParams(collective_id=0),
    )(
        pltpu.with_memory_space_constraint(x, pltpu.MemorySpace.HBM),
        pltpu.with_memory_space_constraint(w_mat, pltpu.MemorySpace.HBM),
    )
